# Initial kernel scaffold; baseline (speedup 1.0000x reference)
import jax
import jax.numpy as jnp
from jax.experimental import pallas as pl


def kernel(x, edge_index, W, b):
    raise NotImplementedError("write your pallas kernel here")


# speedup vs baseline: 8.7454x; 8.7454x over previous
import functools

import jax
import jax.numpy as jnp
from jax import lax
from jax.experimental import pallas as pl
from jax.experimental.pallas import tpu as pltpu
from jax.experimental.pallas import tpu_sc as plsc

N = 10000
E = 160000
D = 256
DH = 128
N_PAD = 10240
E_PAD = 163840
CHUNK = 128
NT = 16
ROWS_T = N_PAD // NT
EROWS = E_PAD // CHUNK
EROWS_T3 = EROWS // NT
EROWS_T1 = EROWS // (2 * NT)

_mesh = plsc.VectorSubcoreMesh(core_axis_name="c", subcore_axis_name="s")


@functools.partial(
    pl.kernel,
    mesh=_mesh,
    out_type=jax.ShapeDtypeStruct((2 * N_PAD,), jnp.float32),
    scratch_types=[
        pltpu.VMEM((EROWS_T1, CHUNK), jnp.int32),
        pltpu.VMEM((CHUNK,), jnp.float32),
        pltpu.VMEM((ROWS_T,), jnp.float32),
        pltpu.VMEM_SHARED((N_PAD,), jnp.float32),
    ],
)
def _deg_call(dst_hbm, out_hbm, idx_v, ones_v, zeros_v, deg_sh):
    c = lax.axis_index("c")
    s = lax.axis_index("s")

    row0 = c * (NT * EROWS_T1) + s * EROWS_T1
    pltpu.sync_copy(dst_hbm.at[pl.ds(row0, EROWS_T1)], idx_v)

    for i in range(CHUNK // 16):
        ones_v[pl.ds(i * 16, 16)] = jnp.full((16,), 1.0, jnp.float32)

    def zbody(i, carry):
        zeros_v[pl.ds(i * 16, 16)] = jnp.zeros((16,), jnp.float32)
        return carry
    lax.fori_loop(0, ROWS_T // 16, zbody, 0)

    pltpu.sync_copy(zeros_v, deg_sh.at[pl.ds(s * ROWS_T, ROWS_T)])
    plsc.subcore_barrier()

    def body(k, carry):
        pltpu.sync_copy(ones_v, deg_sh.at[idx_v.at[k]], add=True)
        return carry
    lax.fori_loop(0, EROWS_T1, body, 0)
    plsc.subcore_barrier()

    pltpu.sync_copy(deg_sh.at[pl.ds(s * ROWS_T, ROWS_T)],
                    out_hbm.at[pl.ds(c * N_PAD + s * ROWS_T, ROWS_T)])


@functools.partial(
    pl.kernel,
    mesh=_mesh,
    out_type=jax.ShapeDtypeStruct((2 * N_PAD, DH), jnp.float32),
    scratch_types=[
        pltpu.VMEM((EROWS_T3, CHUNK), jnp.int32),
        pltpu.VMEM((EROWS_T3, CHUNK), jnp.int32),
        pltpu.VMEM((CHUNK, DH), jnp.float32),
        pltpu.VMEM_SHARED((N_PAD, DH), jnp.float32),
        pltpu.SemaphoreType.DMA,
    ],
)
def _agg_call(y_hbm, src_hbm, dst_hbm, out_hbm, src_v, dst_v, rows_v, acc_sh, sem):
    c = lax.axis_index("c")
    s = lax.axis_index("s")

    r0 = s * ROWS_T
    pltpu.sync_copy(y_hbm.at[pl.ds(c * N_PAD + r0, ROWS_T)],
                    acc_sh.at[pl.ds(r0, ROWS_T)])

    k0 = s * EROWS_T3
    pltpu.sync_copy(src_hbm.at[pl.ds(k0, EROWS_T3)], src_v)
    pltpu.sync_copy(dst_hbm.at[pl.ds(k0, EROWS_T3)], dst_v)

    base = c * N_PAD

    def rebase(i, carry):
        k = i // (CHUNK // 16)
        j = i % (CHUNK // 16)
        sl = pl.ds(j * 16, 16)
        src_v[k, sl] = src_v[k, sl] + base
        return carry
    lax.fori_loop(0, EROWS_T3 * (CHUNK // 16), rebase, 0)
    plsc.subcore_barrier()

    def body(k, carry):
        pltpu.async_copy(y_hbm.at[src_v.at[k]], rows_v, sem).wait()
        pltpu.sync_copy(rows_v, acc_sh.at[dst_v.at[k]], add=True)
        return carry
    lax.fori_loop(0, EROWS_T3, body, 0)
    plsc.subcore_barrier()

    pltpu.sync_copy(acc_sh.at[pl.ds(r0, ROWS_T)],
                    out_hbm.at[pl.ds(c * N_PAD + r0, ROWS_T)])


_RB = 512


def _mm_body(x_ref, w_ref, dga_ref, dgb_ref, y_ref):
    dinv = lax.rsqrt(dga_ref[...] + dgb_ref[...] + 1.0)
    acc = jnp.dot(x_ref[...], w_ref[...], preferred_element_type=jnp.float32)
    y_ref[...] = acc * dinv[:, None]


def _mm_call(x_pad, w, dga, dgb):
    nb = N_PAD // _RB
    return pl.pallas_call(
        _mm_body,
        grid=(nb, 2),
        in_specs=[
            pl.BlockSpec((_RB, D), lambda i, h: (i, 0)),
            pl.BlockSpec((D, DH), lambda i, h: (0, h)),
            pl.BlockSpec((_RB,), lambda i, h: (i,)),
            pl.BlockSpec((_RB,), lambda i, h: (i,)),
        ],
        out_specs=pl.BlockSpec((_RB, DH), lambda i, h: (h * nb + i, 0)),
        out_shape=jax.ShapeDtypeStruct((2 * N_PAD, DH), jnp.float32),
    )(x_pad, w, dga, dgb)


def _fin_body(a_ref, dga_ref, dgb_ref, b_ref, o_ref):
    dinv = lax.rsqrt(dga_ref[...] + dgb_ref[...] + 1.0)
    o_ref[...] = jnp.tanh(a_ref[0] * dinv[:, None] + b_ref[...][None, :])


def _fin_call(a3, dga, dgb, b):
    nb = N_PAD // _RB
    return pl.pallas_call(
        _fin_body,
        grid=(nb, 2),
        in_specs=[
            pl.BlockSpec((1, _RB, DH), lambda i, h: (h, i, 0)),
            pl.BlockSpec((_RB,), lambda i, h: (i,)),
            pl.BlockSpec((_RB,), lambda i, h: (i,)),
            pl.BlockSpec((DH,), lambda i, h: (h,)),
        ],
        out_specs=pl.BlockSpec((_RB, DH), lambda i, h: (i, h)),
        out_shape=jax.ShapeDtypeStruct((N, D), jnp.float32),
    )(a3, dga, dgb, b)


def kernel(x, edge_index, W, b):
    x = x.astype(jnp.float32)
    src = edge_index[0].astype(jnp.int32)
    dst = edge_index[1].astype(jnp.int32)

    npe = E_PAD - E
    pad_src = jnp.zeros((npe,), jnp.int32)
    pad_dst = N + (jnp.arange(npe, dtype=jnp.int32) % (N_PAD - N))
    src2 = jnp.concatenate([src, pad_src]).reshape(EROWS, CHUNK)
    dst2 = jnp.concatenate([dst, pad_dst]).reshape(EROWS, CHUNK)
    x_pad = jnp.pad(x, ((0, N_PAD - N), (0, 0)))

    deg2 = _deg_call(dst2)
    dga, dgb = deg2[:N_PAD], deg2[N_PAD:]
    y2 = _mm_call(x_pad, W, dga, dgb)
    a2 = _agg_call(y2, src2, dst2)
    return _fin_call(a2.reshape(2, N_PAD, DH), dga, dgb, b)


# speedup vs baseline: 10.3147x; 1.1794x over previous
import functools

import jax
import jax.numpy as jnp
from jax import lax
from jax.experimental import pallas as pl
from jax.experimental.pallas import tpu as pltpu
from jax.experimental.pallas import tpu_sc as plsc

N = 10000
E = 160000
D = 256
DH = 128
N_PAD = 10240
E_PAD = 163840
CHUNK = 128
NT = 16
ROWS_T = N_PAD // NT
EROWS = E_PAD // CHUNK
EROWS_T3 = EROWS // NT
EROWS_T1 = EROWS // (2 * NT)

_mesh = plsc.VectorSubcoreMesh(core_axis_name="c", subcore_axis_name="s")


@functools.partial(
    pl.kernel,
    mesh=_mesh,
    out_type=jax.ShapeDtypeStruct((2 * N_PAD,), jnp.float32),
    scratch_types=[
        pltpu.VMEM((EROWS_T1, CHUNK), jnp.int32),
        pltpu.VMEM((CHUNK,), jnp.float32),
        pltpu.VMEM((ROWS_T,), jnp.float32),
        pltpu.VMEM_SHARED((N_PAD,), jnp.float32),
    ],
)
def _deg_call(dst_hbm, out_hbm, idx_v, ones_v, zeros_v, deg_sh):
    c = lax.axis_index("c")
    s = lax.axis_index("s")

    row0 = c * (NT * EROWS_T1) + s * EROWS_T1
    pltpu.sync_copy(dst_hbm.at[pl.ds(row0, EROWS_T1)], idx_v)

    for i in range(CHUNK // 16):
        ones_v[pl.ds(i * 16, 16)] = jnp.full((16,), 1.0, jnp.float32)

    def zbody(i, carry):
        zeros_v[pl.ds(i * 16, 16)] = jnp.zeros((16,), jnp.float32)
        return carry
    lax.fori_loop(0, ROWS_T // 16, zbody, 0)

    pltpu.sync_copy(zeros_v, deg_sh.at[pl.ds(s * ROWS_T, ROWS_T)])
    plsc.subcore_barrier()

    def body(k, carry):
        pltpu.sync_copy(ones_v, deg_sh.at[idx_v.at[k]], add=True)
        return carry
    lax.fori_loop(0, EROWS_T1, body, 0)
    plsc.subcore_barrier()

    pltpu.sync_copy(deg_sh.at[pl.ds(s * ROWS_T, ROWS_T)],
                    out_hbm.at[pl.ds(c * N_PAD + s * ROWS_T, ROWS_T)])


_NR = 2
_NI = 4


@functools.partial(
    pl.kernel,
    mesh=_mesh,
    out_type=jax.ShapeDtypeStruct((2 * N_PAD, DH), jnp.float32),
    scratch_types=[
        pltpu.VMEM((_NI, CHUNK), jnp.int32),
        pltpu.VMEM((_NI, CHUNK), jnp.int32),
        pltpu.VMEM((_NR, CHUNK, DH), jnp.float32),
        pltpu.VMEM_SHARED((N_PAD, DH), jnp.float32),
    ] + [pltpu.SemaphoreType.DMA] * (2 * _NR + _NI),
)
def _agg_call(y_hbm, src_hbm, dst_hbm, out_hbm, sidx_v, didx_v, rows_v, acc_sh,
              *sems):
    c = lax.axis_index("c")
    s = lax.axis_index("s")
    gsem = sems[:_NR]
    ssem = sems[_NR:2 * _NR]
    isem = sems[2 * _NR:]

    r0 = s * ROWS_T
    pltpu.sync_copy(y_hbm.at[pl.ds(c * N_PAD + r0, ROWS_T)],
                    acc_sh.at[pl.ds(r0, ROWS_T)])
    plsc.subcore_barrier()

    k0 = s * EROWS_T3
    sk0 = c * EROWS + k0

    def istart(row, slot):
        pltpu.async_copy(src_hbm.at[sk0 + row], sidx_v.at[slot], isem[slot])
        pltpu.async_copy(dst_hbm.at[k0 + row], didx_v.at[slot], isem[slot])

    def iwait(slot):
        pltpu.make_async_copy(src_hbm.at[0], sidx_v.at[slot], isem[slot]).wait()
        pltpu.make_async_copy(dst_hbm.at[0], didx_v.at[slot], isem[slot]).wait()

    def gstart(islot, slot):
        pltpu.async_copy(y_hbm.at[sidx_v.at[islot]], rows_v.at[slot],
                         gsem[slot])

    def gwait(slot):
        pltpu.make_async_copy(y_hbm.at[sidx_v.at[0]], rows_v.at[slot],
                              gsem[slot]).wait()

    def sstart(islot, slot):
        pltpu.async_copy(rows_v.at[slot], acc_sh.at[didx_v.at[islot]],
                         ssem[slot], add=True)

    def swait(slot):
        pltpu.make_async_copy(rows_v.at[slot], acc_sh.at[didx_v.at[0]],
                              ssem[slot]).wait()

    istart(0, 0)
    istart(1, 1)
    istart(2, 2)
    iwait(0)
    gstart(0, 0)

    NG = EROWS_T3 // 4

    def outer(g, carry):
        for b in range(4):
            sl_k = b % 2
            sl_k1 = (b + 1) % 2
            il_k = b % _NI
            il_k1 = (b + 1) % _NI
            il_k3 = (b + 3) % _NI

            if b == 0:
                @pl.when(g > 0)
                def _():
                    swait(sl_k1)
            else:
                swait(sl_k1)

            if b == 0:
                istart(g * 4 + b + 3, il_k3)
            else:
                @pl.when(g < NG - 1)
                def _():
                    istart(g * 4 + b + 3, il_k3)

            if b < 3:
                iwait(il_k1)
                gstart(il_k1, sl_k1)
            else:
                @pl.when(g < NG - 1)
                def _():
                    iwait(il_k1)
                    gstart(il_k1, sl_k1)

            gwait(sl_k)
            sstart(il_k, sl_k)
        return carry
    lax.fori_loop(0, NG, outer, 0)
    swait((EROWS_T3 - 1) % 2)
    plsc.subcore_barrier()

    pltpu.sync_copy(acc_sh.at[pl.ds(r0, ROWS_T)],
                    out_hbm.at[pl.ds(c * N_PAD + r0, ROWS_T)])


_RB = 512


def _mm_body(x_ref, w_ref, dga_ref, dgb_ref, y_ref):
    dinv = lax.rsqrt(dga_ref[...] + dgb_ref[...] + 1.0)
    acc = jnp.dot(x_ref[...], w_ref[...], preferred_element_type=jnp.float32)
    y_ref[...] = acc * dinv[:, None]


def _mm_call(x_pad, w, dga, dgb):
    nb = N_PAD // _RB
    return pl.pallas_call(
        _mm_body,
        grid=(nb, 2),
        in_specs=[
            pl.BlockSpec((_RB, D), lambda i, h: (i, 0)),
            pl.BlockSpec((D, DH), lambda i, h: (0, h)),
            pl.BlockSpec((_RB,), lambda i, h: (i,)),
            pl.BlockSpec((_RB,), lambda i, h: (i,)),
        ],
        out_specs=pl.BlockSpec((_RB, DH), lambda i, h: (h * nb + i, 0)),
        out_shape=jax.ShapeDtypeStruct((2 * N_PAD, DH), jnp.float32),
    )(x_pad, w, dga, dgb)


def _fin_body(a_ref, dga_ref, dgb_ref, b_ref, o_ref):
    dinv = lax.rsqrt(dga_ref[...] + dgb_ref[...] + 1.0)
    o_ref[...] = jnp.tanh(a_ref[0] * dinv[:, None] + b_ref[...][None, :])


def _fin_call(a3, dga, dgb, b):
    nb = N_PAD // _RB
    return pl.pallas_call(
        _fin_body,
        grid=(nb, 2),
        in_specs=[
            pl.BlockSpec((1, _RB, DH), lambda i, h: (h, i, 0)),
            pl.BlockSpec((_RB,), lambda i, h: (i,)),
            pl.BlockSpec((_RB,), lambda i, h: (i,)),
            pl.BlockSpec((DH,), lambda i, h: (h,)),
        ],
        out_specs=pl.BlockSpec((_RB, DH), lambda i, h: (i, h)),
        out_shape=jax.ShapeDtypeStruct((N, D), jnp.float32),
    )(a3, dga, dgb, b)


def kernel(x, edge_index, W, b):
    x = x.astype(jnp.float32)
    src = edge_index[0].astype(jnp.int32)
    dst = edge_index[1].astype(jnp.int32)

    npe = E_PAD - E
    pad_src = jnp.zeros((npe,), jnp.int32)
    pad_dst = N + (jnp.arange(npe, dtype=jnp.int32) % (N_PAD - N))
    src2 = jnp.concatenate([src, pad_src]).reshape(EROWS, CHUNK)
    dst2 = jnp.concatenate([dst, pad_dst]).reshape(EROWS, CHUNK)
    src2c = jnp.concatenate([src2, src2 + N_PAD], axis=0)
    x_pad = jnp.pad(x, ((0, N_PAD - N), (0, 0)))

    deg2 = _deg_call(dst2)
    dga, dgb = deg2[:N_PAD], deg2[N_PAD:]
    y2 = _mm_call(x_pad, W, dga, dgb)
    a2 = _agg_call(y2, src2c, dst2)
    return _fin_call(a2.reshape(2, N_PAD, DH), dga, dgb, b)
